# Initial kernel scaffold; baseline (speedup 1.0000x reference)
import jax
import jax.numpy as jnp
from jax.experimental import pallas as pl


def kernel(h, coords, height, width, slide_label, W1, b1, W2a, b2a, W2b, b2b, W3, b3, Wc, bc, eps):
    raise NotImplementedError("write your pallas kernel here")


# speedup vs baseline: 1.9790x; 1.9790x over previous
import functools

import numpy as np
import jax
import jax.numpy as jnp
from jax import lax
from jax.experimental import pallas as pl
from jax.experimental.pallas import tpu as pltpu
from jax.experimental.pallas import tpu_sc as plsc

PATCH = 256
GH = GW = 256
GN = GH * GW
NC = 2
NS = 16
NW = NC * NS
L = 16

ROWS = 256


def _gauss_weights():
    ax = np.arange(3, dtype=np.float32)
    g = np.exp(-((ax - 1.0) / 0.5) ** 2 / 2.0) / (0.5 * np.sqrt(2.0 * np.pi))
    k = np.outer(g, g)
    return (k / k.sum()).astype(np.float32)


def _mlp_body(h_ref, w1_ref, b1_ref, w2a_ref, b2a_ref, w2b_ref, b2b_ref,
              w3_ref, b3_ref, h1_ref, pt_ref):
    h = h_ref[...]
    h1 = lax.dot_general(h, w1_ref[...], (((1,), (1,)), ((), ())),
                         preferred_element_type=jnp.float32)
    h1 = jnp.maximum(h1 + b1_ref[...], 0.0)
    za = lax.dot_general(h1, w2a_ref[...], (((1,), (1,)), ((), ())),
                         preferred_element_type=jnp.float32) + b2a_ref[...]
    zb = lax.dot_general(h1, w2b_ref[...], (((1,), (1,)), ((), ())),
                         preferred_element_type=jnp.float32) + b2b_ref[...]
    feat = jax.nn.sigmoid(za) * jnp.tanh(zb)
    pt = lax.dot_general(w3_ref[...], feat, (((1,), (1,)), ((), ())),
                         preferred_element_type=jnp.float32) + b3_ref[...]
    h1_ref[...] = h1
    pt_ref[...] = pt


def _run_mlp(h, W1, b1, W2a, b2a, W2b, b2b, W3, b3):
    n, d_in = h.shape
    d1 = W1.shape[0]
    d2 = W2a.shape[0]
    grid = n // ROWS
    w3p = jnp.zeros((8, d2), jnp.float32).at[:2].set(W3)
    b3p = jnp.zeros((8, 1), jnp.float32).at[:2, 0].set(b3)
    return pl.pallas_call(
        _mlp_body,
        grid=(grid,),
        in_specs=[
            pl.BlockSpec((ROWS, d_in), lambda i: (i, 0)),
            pl.BlockSpec((d1, d_in), lambda i: (0, 0)),
            pl.BlockSpec((1, d1), lambda i: (0, 0)),
            pl.BlockSpec((d2, d1), lambda i: (0, 0)),
            pl.BlockSpec((1, d2), lambda i: (0, 0)),
            pl.BlockSpec((d2, d1), lambda i: (0, 0)),
            pl.BlockSpec((1, d2), lambda i: (0, 0)),
            pl.BlockSpec((8, d2), lambda i: (0, 0)),
            pl.BlockSpec((8, 1), lambda i: (0, 0)),
        ],
        out_specs=[
            pl.BlockSpec((ROWS, d1), lambda i: (i, 0)),
            pl.BlockSpec((8, ROWS), lambda i: (0, i)),
        ],
        out_shape=[
            jax.ShapeDtypeStruct((n, d1), jnp.float32),
            jax.ShapeDtypeStruct((8, n), jnp.float32),
        ],
    )(h, W1, b1.reshape(1, d1), W2a, b2a.reshape(1, d2),
      W2b, b2b.reshape(1, d2), w3p, b3p)


def _scatter_body(ct_hbm, pt_hbm, mu_hbm, lv_hbm,
                  xs_v, ys_v, mup_v, lvp_v, mu_loc, lv_loc):
    n = xs_v.shape[0]
    cells = mu_loc.shape[0]
    wid = lax.axis_index("s") * NC + lax.axis_index("c")
    base = wid * cells

    pltpu.sync_copy(ct_hbm.at[0], xs_v)
    pltpu.sync_copy(ct_hbm.at[1], ys_v)
    pltpu.sync_copy(pt_hbm.at[0], mup_v)
    pltpu.sync_copy(pt_hbm.at[1], lvp_v)

    zeros = jnp.zeros((L,), jnp.float32)

    @pl.loop(0, cells, step=L)
    def _zero(o):
        mu_loc[pl.ds(o, L)] = zeros
        lv_loc[pl.ds(o, L)] = zeros

    @pl.loop(0, n, step=L)
    def _scan(s):
        xs = xs_v[pl.ds(s, L)]
        ys = ys_v[pl.ds(s, L)]
        lin = lax.shift_left(lax.shift_right_logical(ys, 8), 8) \
            + lax.shift_right_logical(xs, 8)
        off = lin - base
        m = (off >= 0) & (off < cells)
        off_c = jnp.where(m, off, 0)
        plsc.store_scatter(mu_loc, [off_c], mup_v[pl.ds(s, L)], mask=m)
        plsc.store_scatter(lv_loc, [off_c], lvp_v[pl.ds(s, L)], mask=m)

    pltpu.sync_copy(mu_loc, mu_hbm.at[pl.ds(base, cells)])
    pltpu.sync_copy(lv_loc, lv_hbm.at[pl.ds(base, cells)])


def _run_scatter(ct, pt):
    n = ct.shape[1]
    cells = GN // NW
    mesh = plsc.VectorSubcoreMesh(core_axis_name="c", subcore_axis_name="s",
                                  num_cores=NC, num_subcores=NS)
    return pl.kernel(
        _scatter_body,
        out_type=[jax.ShapeDtypeStruct((GN,), jnp.float32),
                  jax.ShapeDtypeStruct((GN,), jnp.float32)],
        mesh=mesh,
        scratch_types=[
            pltpu.VMEM((n,), jnp.int32),
            pltpu.VMEM((n,), jnp.int32),
            pltpu.VMEM((n,), jnp.float32),
            pltpu.VMEM((n,), jnp.float32),
            pltpu.VMEM((cells,), jnp.float32),
            pltpu.VMEM((cells,), jnp.float32),
        ],
        compiler_params=pltpu.CompilerParams(needs_layout_passes=False),
    )(ct, pt)


def _grid_body(mu_ref, lv_ref, eps_ref, mupr_ref, lvpr_ref, kl_ref, a_ref):
    w = _gauss_weights()
    mu = mu_ref[...]
    lv = lv_ref[...]
    mu_pr = mupr_ref[0, 0]
    lv_pr = lvpr_ref[0, 0]
    kl_ref[...] = ((lv_pr - lv) / 2.0
                   + (lv * lv + (mu_pr - mu) ** 2) / (2.0 * lv_pr * lv_pr)
                   - 0.5)
    mup = jnp.pad(mu, ((1, 1), (1, 1)))
    mu_s = jnp.zeros_like(mu)
    for dy in range(3):
        for dx in range(3):
            mu_s = mu_s + w[dy, dx] * lax.slice(mup, (dy, dx), (dy + GH, dx + GW))
    a_ref[...] = jax.nn.sigmoid(mu_s + eps_ref[...] * jnp.exp(0.5 * lv))


def _run_grid(mu_g, lv_g, eps2, mu_pr, lv_pr):
    return pl.pallas_call(
        _grid_body,
        out_shape=[jax.ShapeDtypeStruct((GH, GW), jnp.float32),
                   jax.ShapeDtypeStruct((GH, GW), jnp.float32)],
    )(mu_g, lv_g, eps2, mu_pr, lv_pr)


def _gather_body(ct_hbm, a_hbm, pa_hbm, a_loc, xs_v, ys_v, pa_v):
    chunk = xs_v.shape[0]
    wid = lax.axis_index("s") * NC + lax.axis_index("c")
    cbase = wid * chunk

    pltpu.sync_copy(a_hbm, a_loc)
    pltpu.sync_copy(ct_hbm.at[0, pl.ds(cbase, chunk)], xs_v)
    pltpu.sync_copy(ct_hbm.at[1, pl.ds(cbase, chunk)], ys_v)

    @pl.loop(0, chunk, step=L)
    def _gather(s):
        xs = xs_v[pl.ds(s, L)]
        ys = ys_v[pl.ds(s, L)]
        lin = lax.shift_left(lax.shift_right_logical(ys, 8), 8) \
            + lax.shift_right_logical(xs, 8)
        pa_v[pl.ds(s, L)] = plsc.load_gather(a_loc, [lin])

    pltpu.sync_copy(pa_v, pa_hbm.at[pl.ds(cbase, chunk)])


def _run_gather(ct, a_grid):
    n = ct.shape[1]
    chunk = n // NW
    mesh = plsc.VectorSubcoreMesh(core_axis_name="c", subcore_axis_name="s",
                                  num_cores=NC, num_subcores=NS)
    return pl.kernel(
        _gather_body,
        out_type=jax.ShapeDtypeStruct((n,), jnp.float32),
        mesh=mesh,
        scratch_types=[
            pltpu.VMEM((GN,), jnp.float32),
            pltpu.VMEM((chunk,), jnp.int32),
            pltpu.VMEM((chunk,), jnp.int32),
            pltpu.VMEM((chunk,), jnp.float32),
        ],
        compiler_params=pltpu.CompilerParams(needs_layout_passes=False),
    )(ct, a_grid)


def _head_body(pa_ref, h1_ref, wc_ref, bc_ref, logit_ref, prob_ref, yhat_ref,
               acc_ref, ssum_ref):
    i = pl.program_id(0)
    nsteps = pl.num_programs(0)

    @pl.when(i == 0)
    def _init():
        acc_ref[...] = jnp.zeros_like(acc_ref)
        ssum_ref[0, 0] = 0.0

    a = pa_ref[0]
    hb = h1_ref[...]
    acc_ref[...] += lax.dot_general(a, hb, (((1,), (0,)), ((), ())),
                                    preferred_element_type=jnp.float32)
    ssum_ref[0, 0] += jnp.sum(a)

    @pl.when(i == nsteps - 1)
    def _final():
        m = acc_ref[...] / ssum_ref[0, 0]
        logits = lax.dot_general(m, wc_ref[...], (((1,), (1,)), ((), ())),
                                 preferred_element_type=jnp.float32) + bc_ref[...]
        mx = jnp.max(logits, axis=1, keepdims=True)
        e = jnp.exp(logits - mx)
        probs = e / jnp.sum(e, axis=1, keepdims=True)
        logit_ref[...] = logits
        prob_ref[...] = probs
        yhat_ref[...] = jnp.where(logits[0:1, 1:2] > logits[0:1, 0:1], 1, 0
                                  ).astype(jnp.int32)


def _run_head(pa, h1, Wc, bc):
    n, d1 = h1.shape
    grid = n // ROWS
    pa2 = pa.reshape(grid, 1, ROWS)
    return pl.pallas_call(
        _head_body,
        grid=(grid,),
        in_specs=[
            pl.BlockSpec((1, 1, ROWS), lambda i: (i, 0, 0)),
            pl.BlockSpec((ROWS, d1), lambda i: (i, 0)),
            pl.BlockSpec((2, d1), lambda i: (0, 0)),
            pl.BlockSpec((1, 2), lambda i: (0, 0)),
        ],
        out_specs=[
            pl.BlockSpec((1, 2), lambda i: (0, 0)),
            pl.BlockSpec((1, 2), lambda i: (0, 0)),
            pl.BlockSpec((1, 1), lambda i: (0, 0)),
        ],
        out_shape=[
            jax.ShapeDtypeStruct((1, 2), jnp.float32),
            jax.ShapeDtypeStruct((1, 2), jnp.float32),
            jax.ShapeDtypeStruct((1, 1), jnp.int32),
        ],
        scratch_shapes=[
            pltpu.VMEM((1, d1), jnp.float32),
            pltpu.SMEM((1, 1), jnp.float32),
        ],
    )(pa2, h1, Wc, bc.reshape(1, 2))


def kernel(h, coords, height, width, slide_label, W1, b1, W2a, b2a, W2b, b2b,
           W3, b3, Wc, bc, eps):
    n = h.shape[0]
    h1, pt = _run_mlp(h, W1, b1, W2a, b2a, W2b, b2b, W3, b3)
    ct = coords.T
    mu_g, lv_g = _run_scatter(ct, pt)
    lbl = slide_label[0]
    mu_pr = jnp.where(lbl == 0, -5.0, 0.0).astype(jnp.float32).reshape(1, 1)
    lv_pr = jnp.where(lbl == 0, -1.0, 3.0).astype(jnp.float32).reshape(1, 1)
    kl, a_grid = _run_grid(mu_g.reshape(GH, GW), lv_g.reshape(GH, GW),
                           eps.reshape(GH, GW), mu_pr, lv_pr)
    pa = _run_gather(ct, a_grid.reshape(GN))
    top_instance, y_prob, y_hat = _run_head(pa, h1, Wc, bc)
    return (top_instance, y_prob, y_hat, kl.reshape(1, GH, GW), y_prob,
            pa.reshape(1, n))
